# baseline (speedup 1.0000x reference)
import functools

import jax
import jax.numpy as jnp
import numpy as np
from jax.experimental import pallas as pl
from jax.experimental.pallas import tpu as pltpu

N_NODES = 10000
HID = 128
NEG = 0.2


def _leaky(v, slope):
    return jnp.where(v >= 0, v, slope * v)


def _mm_body(a_ref, b_ref, o_ref):
    o_ref[...] = jnp.dot(a_ref[...], b_ref[...],
                         preferred_element_type=jnp.float32)


def _mm(a, b, bm=400):
    M, K = a.shape
    K2, N = b.shape
    assert K == K2 and M % bm == 0
    return pl.pallas_call(
        _mm_body,
        grid=(M // bm,),
        in_specs=[pl.BlockSpec((bm, K), lambda i: (i, 0)),
                  pl.BlockSpec((K, N), lambda i: (0, 0))],
        out_specs=pl.BlockSpec((bm, N), lambda i: (i, 0)),
        out_shape=jax.ShapeDtypeStruct((M, N), jnp.float32),
    )(a, b)


def _encode(x, edge_index, edge_attr, W_node, W_edge, W_lay, a_src, a_dst, a_edge):
    N = x.shape[0]
    h = _mm(x, W_node)
    e = edge_attr @ W_edge
    src = edge_index[0]
    dst = edge_index[1]
    pe = e @ a_edge.T
    for l in range(W_lay.shape[0]):
        hs = _mm(h, W_lay[l])
        ps = hs @ a_src[l]
        pd = hs @ a_dst[l]
        sc = _leaky(ps[src] + pd[dst] + pe[:, l], 0.2)
        m = jax.ops.segment_max(sc, dst, num_segments=N)
        m = jnp.where(jnp.isfinite(m), m, 0.0)
        ex = jnp.exp(sc - m[dst])
        den = jax.ops.segment_sum(ex, dst, num_segments=N)
        alpha = ex / (den[dst] + 1e-16)
        agg = jax.ops.segment_sum(alpha[:, None] * hs[src], dst, num_segments=N)
        h = h + _leaky(agg, NEG)
    return h


B = 100
NN = 100
SCALE = 1.0 / np.sqrt(128.0)


def _dec_body(h3_ref, g_ref, keys_ref, dem_ref, cap0_ref, wctx_ref, T_ref,
              acts_ref, lps_ref):
    h3 = h3_ref[...]
    g = g_ref[...]
    keys = keys_ref[...]
    dem = dem_ref[...]
    wctx = wctx_ref[...]
    Tval = T_ref[0, 0]
    col = jax.lax.broadcasted_iota(jnp.int32, (B, NN), 1)

    def body(t, carry):
        visited, cur, cap, acts, lps = carry
        ctx = jnp.concatenate([g, cur, cap], axis=-1)
        q = jnp.dot(ctx, wctx, preferred_element_type=jnp.float32)
        sc = jax.lax.dot_general(q, keys, (((1,), (2,)), ((0,), (0,))),
                                 preferred_element_type=jnp.float32) * SCALE
        sc = 10.0 * jnp.tanh(sc) / Tval
        sc = jnp.where(visited > 0, -1e9, sc)
        mx = jnp.max(sc, axis=-1, keepdims=True)
        sh = sc - mx
        lp = sh - jnp.log(jnp.sum(jnp.exp(sh), axis=-1, keepdims=True))
        lmax = jnp.max(lp, axis=-1, keepdims=True)
        a = jnp.min(jnp.where(lp == lmax, col, NN), axis=-1, keepdims=True)
        onehot = (col == a).astype(jnp.float32)
        acts = jnp.where(col == t, a, acts)
        lps_sel = jnp.sum(lp * onehot, axis=-1, keepdims=True)
        lps = jnp.where(col == t, lps_sel, lps)
        visited = jnp.maximum(visited, onehot)
        dsel = jnp.sum(dem * onehot, axis=-1, keepdims=True)
        cap = jnp.maximum(cap - dsel, 0.0)
        cur = jnp.sum(h3 * onehot[:, :, None], axis=1)
        return (visited, cur, cap, acts, lps)

    visited0 = jnp.zeros((B, NN), jnp.float32)
    cur0 = h3[:, 0, :]
    cap0 = cap0_ref[...]
    acts0 = jnp.zeros((B, NN), jnp.int32)
    lps0 = jnp.zeros((B, NN), jnp.float32)
    _, _, _, acts, lps = jax.lax.fori_loop(
        0, NN, body, (visited0, cur0, cap0, acts0, lps0))
    acts_ref[...] = acts
    lps_ref[...] = lps


def _decode_pallas(h, g, dem, cap0, T, W_k, W_ctx):
    h3 = h.reshape(B, NN, HID)
    keys = _mm(h, W_k).reshape(B, NN, HID)
    Tarr = jnp.full((1, 1), T, jnp.float32)
    return pl.pallas_call(
        _dec_body,
        out_shape=(jax.ShapeDtypeStruct((B, NN), jnp.int32),
                   jax.ShapeDtypeStruct((B, NN), jnp.float32)),
    )(h3, g, keys, dem, cap0, W_ctx, Tarr)


def kernel(x, edge_index, edge_attr, batch, demand, capacity, n_steps, greedy,
           T, W_node, W_edge, W_lay, a_src, a_dst, a_edge, W_k, W_ctx):
    h = _encode(x, edge_index, edge_attr, W_node, W_edge, W_lay,
                a_src, a_dst, a_edge)
    Bsz = capacity.shape[0]
    g = h.reshape(Bsz, -1, h.shape[-1]).mean(axis=1)
    dem = jnp.where(batch >= 0, demand, 0.0).reshape(Bsz, -1)
    cap0 = capacity.reshape(Bsz, -1)[:, :1]
    actions, log_p = _decode_pallas(h, g, dem, cap0, T, W_k, W_ctx)
    return (actions, log_p)


# speedup vs baseline: 10.1925x; 10.1925x over previous
import functools

import jax
import jax.numpy as jnp
import numpy as np
from jax import lax
from jax.experimental import pallas as pl
from jax.experimental.pallas import tpu as pltpu
from jax.experimental.pallas import tpu_sc as plsc

N_NODES = 10000
HID = 128
NEG = 0.2
B = 100
NN = 100
SCALE = 1.0 / np.sqrt(128.0)

_E = 160000
_N = 10000
_NP = 10240
_NEG_INF = -3.4028234663852886e38

_mesh = plsc.VectorSubcoreMesh(core_axis_name="c", subcore_axis_name="s")
_SC_PARAMS = pltpu.CompilerParams(needs_layout_passes=False,
                                  use_tc_tiling_on_sc=False)


def _iota16():
    return lax.broadcasted_iota(jnp.int32, (16,), 0)


def _ids():
    cid = lax.axis_index("c")
    sid = lax.axis_index("s")
    return cid, sid, cid * 16 + sid


@functools.partial(
    pl.kernel,
    out_type=[jax.ShapeDtypeStruct((_E,), jnp.float32),
              jax.ShapeDtypeStruct((2 * _NP,), jnp.float32)],
    scratch_types=[
        pltpu.VMEM((5008,), jnp.int32),
        pltpu.VMEM((5008,), jnp.int32),
        pltpu.VMEM((5008,), jnp.float32),
        pltpu.VMEM((5008,), jnp.float32),
        pltpu.VMEM((2 * _NP,), jnp.float32),
        pltpu.VMEM((_NP,), jnp.float32),
        pltpu.VMEM((640,), jnp.float32),
        pltpu.VMEM((640,), jnp.float32),
        pltpu.VMEM((16,), jnp.int32),
        pltpu.VMEM((16,), jnp.float32),
        pltpu.VMEM_SHARED((16, _NP), jnp.float32),
    ],
    mesh=_mesh,
    compiler_params=_SC_PARAMS,
)
def _sc_a(src_hbm, dst_hbm, pe_hbm, psd_hbm, sc_out, mpart_out,
          src_v, dst_v, pe_v, sc_v, psd_v, m_v, tmp_v, acc_v,
          sg_d, sg_v, msh):
    cid, sid, eid = _ids()
    be = (312 * eid + jnp.minimum(eid, 16)) * 16
    it = _iota16()
    pltpu.sync_copy(psd_hbm, psd_v)
    pltpu.sync_copy(src_hbm.at[pl.ds(be, 4992)], src_v.at[pl.ds(0, 4992)])
    pltpu.sync_copy(dst_hbm.at[pl.ds(be, 4992)], dst_v.at[pl.ds(0, 4992)])
    pltpu.sync_copy(pe_hbm.at[pl.ds(be, 4992)], pe_v.at[pl.ds(0, 4992)])

    @pl.when(eid < 16)
    def _():
        pltpu.sync_copy(src_hbm.at[pl.ds(be + 4992, 16)],
                        src_v.at[pl.ds(4992, 16)])
        pltpu.sync_copy(dst_hbm.at[pl.ds(be + 4992, 16)],
                        dst_v.at[pl.ds(4992, 16)])
        pltpu.sync_copy(pe_hbm.at[pl.ds(be + 4992, 16)],
                        pe_v.at[pl.ds(4992, 16)])

    def mi(i, c):
        m_v[pl.ds(i * 16, 16)] = jnp.full((16,), _NEG_INF, jnp.float32)
        return c
    lax.fori_loop(0, 640, mi, 0)

    def group(el):
        s16 = src_v[pl.ds(el, 16)]
        d16 = dst_v[pl.ds(el, 16)]
        ps = plsc.load_gather(psd_v, [s16 * 2])
        pd_ = plsc.load_gather(psd_v, [d16 * 2 + 1])
        x = ps + pd_ + pe_v[pl.ds(el, 16)]
        sc16 = jnp.where(x >= 0, x, 0.2 * x)
        sc_v[pl.ds(el, 16)] = sc16
        dsrt, val = plsc.sort_key_val(d16, sc16)
        sg_d[...] = dsrt
        for k in (1, 2, 4, 8):
            sg_v[...] = val
            idxk = jnp.maximum(it - k, 0)
            pdk = plsc.load_gather(sg_d, [idxk])
            pvk = plsc.load_gather(sg_v, [idxk])
            match = jnp.logical_and(it >= k, pdk == dsrt)
            val = jnp.where(match, jnp.maximum(val, pvk), val)
        nd = plsc.load_gather(sg_d, [jnp.minimum(it + 1, 15)])
        last = jnp.logical_or(it == 15, nd != dsrt)
        old = plsc.load_gather(m_v, [dsrt])
        plsc.store_scatter(m_v, [dsrt], jnp.maximum(old, val), mask=last)

    def gb(g2, c):
        group(g2 * 16)
        return c
    lax.fori_loop(0, 312, gb, 0)

    @pl.when(eid < 16)
    def _():
        group(4992)

    pltpu.sync_copy(sc_v.at[pl.ds(0, 4992)], sc_out.at[pl.ds(be, 4992)])

    @pl.when(eid < 16)
    def _():
        pltpu.sync_copy(sc_v.at[pl.ds(4992, 16)],
                        sc_out.at[pl.ds(be + 4992, 16)])

    pltpu.sync_copy(m_v, msh.at[sid])
    plsc.subcore_barrier()
    nb = sid * 640
    pltpu.sync_copy(msh.at[0, pl.ds(nb, 640)], acc_v)
    for j in range(1, 16):
        pltpu.sync_copy(msh.at[j, pl.ds(nb, 640)], tmp_v)

        def mx(i, c):
            sl = pl.ds(i * 16, 16)
            acc_v[sl] = jnp.maximum(acc_v[sl], tmp_v[sl])
            return c
        lax.fori_loop(0, 40, mx, 0)
    pltpu.sync_copy(acc_v, mpart_out.at[pl.ds(cid * _NP + nb, 640)])


@functools.partial(
    pl.kernel,
    out_type=[jax.ShapeDtypeStruct((_E,), jnp.float32),
              jax.ShapeDtypeStruct((2 * _NP,), jnp.float32)],
    scratch_types=[
        pltpu.VMEM((5008,), jnp.int32),
        pltpu.VMEM((5008,), jnp.float32),
        pltpu.VMEM((5008,), jnp.float32),
        pltpu.VMEM((_NP,), jnp.float32),
        pltpu.VMEM((_NP,), jnp.float32),
        pltpu.VMEM((_NP,), jnp.float32),
        pltpu.VMEM((640,), jnp.float32),
        pltpu.VMEM((640,), jnp.float32),
        pltpu.VMEM((16,), jnp.int32),
        pltpu.VMEM((16,), jnp.float32),
        pltpu.VMEM_SHARED((16, _NP), jnp.float32),
    ],
    mesh=_mesh,
    compiler_params=_SC_PARAMS,
)
def _sc_b(sc_hbm, dst_hbm, mpart_hbm, ex_out, denpart_out,
          dst_v, sc_v, ex_v, m_v, mb_v, den_v, tmp_v, acc_v, sg_d, sg_v,
          densh):
    cid, sid, eid = _ids()
    be = (312 * eid + jnp.minimum(eid, 16)) * 16
    it = _iota16()
    pltpu.sync_copy(dst_hbm.at[pl.ds(be, 4992)], dst_v.at[pl.ds(0, 4992)])
    pltpu.sync_copy(sc_hbm.at[pl.ds(be, 4992)], sc_v.at[pl.ds(0, 4992)])

    @pl.when(eid < 16)
    def _():
        pltpu.sync_copy(dst_hbm.at[pl.ds(be + 4992, 16)],
                        dst_v.at[pl.ds(4992, 16)])
        pltpu.sync_copy(sc_hbm.at[pl.ds(be + 4992, 16)],
                        sc_v.at[pl.ds(4992, 16)])

    pltpu.sync_copy(mpart_hbm.at[pl.ds(0, _NP)], m_v)
    pltpu.sync_copy(mpart_hbm.at[pl.ds(_NP, _NP)], mb_v)

    def mm(i, c):
        sl = pl.ds(i * 16, 16)
        a = jnp.maximum(m_v[sl], mb_v[sl])
        m_v[sl] = jnp.where(a < -3e38, 0.0, a)
        den_v[sl] = jnp.zeros((16,), jnp.float32)
        return c
    lax.fori_loop(0, 640, mm, 0)

    def group(el):
        d16 = dst_v[pl.ds(el, 16)]
        sc16 = sc_v[pl.ds(el, 16)]
        m16 = plsc.load_gather(m_v, [d16])
        ex16 = jnp.exp(sc16 - m16)
        ex_v[pl.ds(el, 16)] = ex16
        dsrt, val = plsc.sort_key_val(d16, ex16)
        sg_d[...] = dsrt
        for k in (1, 2, 4, 8):
            sg_v[...] = val
            idxk = jnp.maximum(it - k, 0)
            pdk = plsc.load_gather(sg_d, [idxk])
            pvk = plsc.load_gather(sg_v, [idxk])
            match = jnp.logical_and(it >= k, pdk == dsrt)
            val = val + jnp.where(match, pvk, 0.0)
        nd = plsc.load_gather(sg_d, [jnp.minimum(it + 1, 15)])
        last = jnp.logical_or(it == 15, nd != dsrt)
        plsc.addupdate_scatter(den_v, [dsrt], val, mask=last)

    def gb(g2, c):
        group(g2 * 16)
        return c
    lax.fori_loop(0, 312, gb, 0)

    @pl.when(eid < 16)
    def _():
        group(4992)

    pltpu.sync_copy(ex_v.at[pl.ds(0, 4992)], ex_out.at[pl.ds(be, 4992)])

    @pl.when(eid < 16)
    def _():
        pltpu.sync_copy(ex_v.at[pl.ds(4992, 16)],
                        ex_out.at[pl.ds(be + 4992, 16)])

    pltpu.sync_copy(den_v, densh.at[sid])
    plsc.subcore_barrier()
    nb = sid * 640
    pltpu.sync_copy(densh.at[0, pl.ds(nb, 640)], acc_v)
    for j in range(1, 16):
        pltpu.sync_copy(densh.at[j, pl.ds(nb, 640)], tmp_v)

        def ad(i, c):
            sl = pl.ds(i * 16, 16)
            acc_v[sl] = acc_v[sl] + tmp_v[sl]
            return c
        lax.fori_loop(0, 40, ad, 0)
    pltpu.sync_copy(acc_v, denpart_out.at[pl.ds(cid * _NP + nb, 640)])


@functools.partial(
    pl.kernel,
    out_type=[jax.ShapeDtypeStruct((2, _NP, 128), jnp.float32)],
    scratch_types=[
        pltpu.VMEM((_NP,), jnp.float32),
        pltpu.VMEM((_NP,), jnp.float32),
        pltpu.VMEM((128,), jnp.int32),
        pltpu.VMEM((1, 128), jnp.int32),
        pltpu.VMEM((128,), jnp.float32),
        pltpu.VMEM((128,), jnp.float32),
        pltpu.VMEM((128, 128), jnp.float32),
        pltpu.SemaphoreType.DMA,
        pltpu.VMEM_SHARED((_NP, 128), jnp.float32),
    ],
    mesh=_mesh,
    compiler_params=_SC_PARAMS,
)
def _sc_c(ex_hbm, src_hbm, dst_hbm, denpart_hbm, hs_hbm, agg_out,
          den_v, db_v, srcrow, dstrow2, exrow, alpharow, rows_v, sem, aggsh):
    cid, sid, eid = _ids()
    rb = 39 * eid + jnp.minimum(eid, 2)
    pltpu.sync_copy(denpart_hbm.at[pl.ds(0, _NP)], den_v)
    pltpu.sync_copy(denpart_hbm.at[pl.ds(_NP, _NP)], db_v)

    def dd(i, c):
        sl = pl.ds(i * 16, 16)
        den_v[sl] = den_v[sl] + db_v[sl] + 1e-16
        return c
    lax.fori_loop(0, 640, dd, 0)

    def zr(i, c):
        for j in range(8):
            rows_v[i, pl.ds(j * 16, 16)] = jnp.zeros((16,), jnp.float32)
        return c
    lax.fori_loop(0, 128, zr, 0)
    for q in range(5):
        pltpu.sync_copy(rows_v, aggsh.at[pl.ds(sid * 640 + q * 128, 128), :])
    plsc.subcore_barrier()

    def row(r):
        g = rb + r
        pltpu.sync_copy(src_hbm.at[pl.ds(g * 128, 128)], srcrow)
        pltpu.sync_copy(dst_hbm.at[pl.ds(g * 128, 128)], dstrow2.at[0])
        pltpu.sync_copy(ex_hbm.at[pl.ds(g * 128, 128)], exrow)
        pltpu.async_copy(hs_hbm.at[srcrow], rows_v, sem).wait()
        for j in range(8):
            sl = pl.ds(j * 16, 16)
            d16 = dstrow2[0, sl]
            den16 = plsc.load_gather(den_v, [d16])
            alpharow[sl] = exrow[sl] / den16

        def sr(rr, c2):
            av = plsc.load_gather(alpharow,
                                  [jnp.full((16,), rr, jnp.int32)])
            for j in range(8):
                sl = pl.ds(j * 16, 16)
                rows_v[rr, sl] = rows_v[rr, sl] * av
            return c2
        lax.fori_loop(0, 128, sr, 0)
        pltpu.sync_copy(rows_v, aggsh.at[dstrow2.at[0]], add=True)

    def rloop(r, c):
        row(r)
        return c
    lax.fori_loop(0, 39, rloop, 0)

    @pl.when(eid < 2)
    def _():
        row(39)

    plsc.subcore_barrier()
    for q in range(5):
        nb = sid * 640 + q * 128
        pltpu.sync_copy(aggsh.at[pl.ds(nb, 128), :], rows_v)
        pltpu.sync_copy(rows_v, agg_out.at[cid, pl.ds(nb, 128), :])


def _enc0_body(x_ref, wn_ref, w_ref, u_ref, h_ref, hs_ref, psd_ref):
    h = jnp.dot(x_ref[...], wn_ref[...], preferred_element_type=jnp.float32)
    h_ref[...] = h
    hs = jnp.dot(h, w_ref[...], preferred_element_type=jnp.float32)
    hs_ref[...] = hs
    psd_ref[...] = jnp.dot(hs, u_ref[...], preferred_element_type=jnp.float32)


def _enc0(x, W_node, W, u):
    bm = 400
    return pl.pallas_call(
        _enc0_body,
        grid=(_N // bm,),
        in_specs=[pl.BlockSpec((bm, 128), lambda i: (i, 0)),
                  pl.BlockSpec((128, 128), lambda i: (0, 0)),
                  pl.BlockSpec((128, 128), lambda i: (0, 0)),
                  pl.BlockSpec((128, 2), lambda i: (0, 0))],
        out_specs=[pl.BlockSpec((bm, 128), lambda i: (i, 0)),
                   pl.BlockSpec((bm, 128), lambda i: (i, 0)),
                   pl.BlockSpec((bm, 2), lambda i: (i, 0))],
        out_shape=[jax.ShapeDtypeStruct((_N, 128), jnp.float32),
                   jax.ShapeDtypeStruct((_N, 128), jnp.float32),
                   jax.ShapeDtypeStruct((_N, 2), jnp.float32)],
    )(x, W_node, W, u)


def _encl_body(h_ref, agg_ref, w_ref, u_ref, hn_ref, hs_ref, psd_ref):
    a = agg_ref[0] + agg_ref[1]
    hn = h_ref[...] + jnp.where(a >= 0, a, NEG * a)
    hn_ref[...] = hn
    hs = jnp.dot(hn, w_ref[...], preferred_element_type=jnp.float32)
    hs_ref[...] = hs
    psd_ref[...] = jnp.dot(hs, u_ref[...], preferred_element_type=jnp.float32)


def _encl(h, aggpart, W, u):
    bm = 400
    return pl.pallas_call(
        _encl_body,
        grid=(_N // bm,),
        in_specs=[pl.BlockSpec((bm, 128), lambda i: (i, 0)),
                  pl.BlockSpec((2, bm, 128), lambda i: (0, i, 0)),
                  pl.BlockSpec((128, 128), lambda i: (0, 0)),
                  pl.BlockSpec((128, 2), lambda i: (0, 0))],
        out_specs=[pl.BlockSpec((bm, 128), lambda i: (i, 0)),
                   pl.BlockSpec((bm, 128), lambda i: (i, 0)),
                   pl.BlockSpec((bm, 2), lambda i: (i, 0))],
        out_shape=[jax.ShapeDtypeStruct((_N, 128), jnp.float32),
                   jax.ShapeDtypeStruct((_N, 128), jnp.float32),
                   jax.ShapeDtypeStruct((_N, 2), jnp.float32)],
    )(h, aggpart, W, u)


def _encode_sc(x, edge_index, edge_attr, W_node, W_edge, W_lay, a_src, a_dst,
               a_edge, W_k):
    src = edge_index[0]
    dst = edge_index[1]
    e = edge_attr @ W_edge
    u = jnp.stack([a_src, a_dst], axis=-1)
    h, hs, psd = _enc0(x, W_node, W_lay[0], u[0])
    keys = None
    for l in range(W_lay.shape[0]):
        pe_l = e @ a_edge[l]
        psd_flat = jnp.pad(psd, ((0, _NP - _N), (0, 0))).reshape(-1)
        scl, mpart = _sc_a(src, dst, pe_l, psd_flat)
        ex, denpart = _sc_b(scl, dst, mpart)
        aggpart, = _sc_c(ex, src, dst, denpart, hs)
        if l < W_lay.shape[0] - 1:
            h, hs, psd = _encl(h, aggpart, W_lay[l + 1], u[l + 1])
        else:
            h, keys, _ = _encl(h, aggpart, W_k,
                               jnp.zeros((128, 2), jnp.float32))
    return h, keys


def _dec_body(h3_ref, g_ref, keys_ref, dem_ref, cap0_ref, wctx_ref, T_ref,
              acts_ref, lps_ref):
    h3 = h3_ref[...]
    g = g_ref[...]
    keys = keys_ref[...]
    dem = dem_ref[...]
    wctx = wctx_ref[...]
    Tval = T_ref[0, 0]
    col = lax.broadcasted_iota(jnp.int32, (B, NN), 1)

    def body(t, carry):
        visited, cur, cap, acts, lps = carry
        ctx = jnp.concatenate([g, cur, cap], axis=-1)
        q = jnp.dot(ctx, wctx, preferred_element_type=jnp.float32)
        sc = lax.dot_general(q, keys, (((1,), (2,)), ((0,), (0,))),
                             preferred_element_type=jnp.float32) * SCALE
        sc = 10.0 * jnp.tanh(sc) / Tval
        sc = jnp.where(visited > 0, -1e9, sc)
        mx = jnp.max(sc, axis=-1, keepdims=True)
        sh = sc - mx
        lp = sh - jnp.log(jnp.sum(jnp.exp(sh), axis=-1, keepdims=True))
        lmax = jnp.max(lp, axis=-1, keepdims=True)
        a = jnp.min(jnp.where(lp == lmax, col, NN), axis=-1, keepdims=True)
        onehot = (col == a).astype(jnp.float32)
        acts = jnp.where(col == t, a, acts)
        lps_sel = jnp.sum(lp * onehot, axis=-1, keepdims=True)
        lps = jnp.where(col == t, lps_sel, lps)
        visited = jnp.maximum(visited, onehot)
        dsel = jnp.sum(dem * onehot, axis=-1, keepdims=True)
        cap = jnp.maximum(cap - dsel, 0.0)
        cur = jnp.sum(h3 * onehot[:, :, None], axis=1)
        return (visited, cur, cap, acts, lps)

    visited0 = jnp.zeros((B, NN), jnp.float32)
    cur0 = h3[:, 0, :]
    cap0 = cap0_ref[...]
    acts0 = jnp.zeros((B, NN), jnp.int32)
    lps0 = jnp.zeros((B, NN), jnp.float32)
    _, _, _, acts, lps = lax.fori_loop(
        0, NN, body, (visited0, cur0, cap0, acts0, lps0))
    acts_ref[...] = acts
    lps_ref[...] = lps


def _decode_pallas(h, keys, g, dem, cap0, T, W_ctx):
    h3 = h.reshape(B, NN, HID)
    keys3 = keys.reshape(B, NN, HID)
    Tarr = jnp.full((1, 1), T, jnp.float32)
    return pl.pallas_call(
        _dec_body,
        out_shape=(jax.ShapeDtypeStruct((B, NN), jnp.int32),
                   jax.ShapeDtypeStruct((B, NN), jnp.float32)),
    )(h3, g, keys3, dem, cap0, W_ctx, Tarr)


def kernel(x, edge_index, edge_attr, batch, demand, capacity, n_steps, greedy,
           T, W_node, W_edge, W_lay, a_src, a_dst, a_edge, W_k, W_ctx):
    h, keys = _encode_sc(x, edge_index, edge_attr, W_node, W_edge, W_lay,
                         a_src, a_dst, a_edge, W_k)
    Bsz = capacity.shape[0]
    g = h.reshape(Bsz, -1, h.shape[-1]).mean(axis=1)
    dem = jnp.where(batch >= 0, demand, 0.0).reshape(Bsz, -1)
    cap0 = capacity.reshape(Bsz, -1)[:, :1]
    actions, log_p = _decode_pallas(h, keys, g, dem, cap0, T, W_ctx)
    return (actions, log_p)
